# baseline (speedup 1.0000x reference)
import functools

import jax
import jax.numpy as jnp
from jax import lax
from jax.experimental import pallas as pl
from jax.experimental.pallas import tpu as pltpu
from jax.experimental.pallas import tpu_sc as plsc

BATCH = 16384
HIST = 50
EMB = 128
B_TOTAL = BATCH * HIST
G = 128
NC = 2
NS = 16
NW = NC * NS
B_PER_W = B_TOTAL // NW
N_CHUNKS = B_PER_W // G
NBUF = 4
S_OUTER = N_CHUNKS // NBUF


@functools.partial(
    pl.kernel,
    out_type=jax.ShapeDtypeStruct((B_TOTAL, EMB), jnp.float32),
    mesh=plsc.VectorSubcoreMesh(core_axis_name="c", subcore_axis_name="s"),
    scratch_types=[
        pltpu.VMEM((B_PER_W,), jnp.int32),
        pltpu.VMEM((NBUF, G, EMB), jnp.float32),
        pltpu.SemaphoreType.DMA((NBUF,)),
        pltpu.SemaphoreType.DMA((NBUF,)),
    ],
)
def _gather_kernel(idx_hbm, table_hbm, out_hbm, idx_v, rows_v, gsem, osem):
  wid = lax.axis_index("s") * NC + lax.axis_index("c")
  base = wid * B_PER_W

  def fire_gather(j, b):
    pltpu.async_copy(table_hbm.at[idx_v.at[pl.ds(j * G, G)]],
                     rows_v.at[b], gsem.at[b])

  def drain_out(b):
    pltpu.make_async_copy(
        out_hbm.at[pl.ds(base, G)], rows_v.at[b], osem.at[b]).wait()

  pltpu.sync_copy(idx_hbm.at[pl.ds(base, B_PER_W)], idx_v)

  for j in range(NBUF - 1):
    fire_gather(j, j)

  def outer(s, carry):
    for b in range(NBUF):
      i = s * NBUF + b
      pltpu.make_async_copy(
          table_hbm.at[idx_v.at[pl.ds(0, G)]], rows_v.at[b], gsem.at[b]).wait()
      pltpu.async_copy(
          rows_v.at[b], out_hbm.at[pl.ds(base + i * G, G)], osem.at[b])
      bn = (b + NBUF - 1) % NBUF
      if b == 0:
        @pl.when(s > 0)
        def _():
          drain_out(bn)
        fire_gather(i + NBUF - 1, bn)
      else:
        @pl.when(s < S_OUTER - 1)
        def _():
          drain_out(bn)
          fire_gather(i + NBUF - 1, bn)
    return carry

  lax.fori_loop(0, S_OUTER, outer, 0)

  for b in range(NBUF):
    drain_out(b)


def kernel(input_, weight):
  idx = input_.T.reshape(B_TOTAL).astype(jnp.int32)
  out = _gather_kernel(idx, weight)
  return out.reshape(HIST, BATCH, EMB).transpose(1, 0, 2)


# speedup vs baseline: 1.0038x; 1.0038x over previous
import functools

import jax
import jax.numpy as jnp
from jax import lax
from jax.experimental import pallas as pl
from jax.experimental.pallas import tpu as pltpu
from jax.experimental.pallas import tpu_sc as plsc

BATCH = 16384
HIST = 50
EMB = 128
B_TOTAL = BATCH * HIST
G = 128
NC = 2
NS = 16
NW = NC * NS
B_PER_W = B_TOTAL // NW
N_CHUNKS = B_PER_W // G
NBUF = 5
S_OUTER = N_CHUNKS // NBUF


@functools.partial(
    pl.kernel,
    out_type=jax.ShapeDtypeStruct((B_TOTAL, EMB), jnp.float32),
    mesh=plsc.VectorSubcoreMesh(core_axis_name="c", subcore_axis_name="s"),
    scratch_types=[
        pltpu.VMEM((B_PER_W,), jnp.int32),
        pltpu.VMEM((NBUF, G, EMB), jnp.float32),
        pltpu.SemaphoreType.DMA((NBUF,)),
        pltpu.SemaphoreType.DMA((NBUF,)),
    ],
)
def _gather_kernel(idx_hbm, table_hbm, out_hbm, idx_v, rows_v, gsem, osem):
  wid = lax.axis_index("s") * NC + lax.axis_index("c")
  base = wid * B_PER_W

  def fire_gather(j, b):
    pltpu.async_copy(table_hbm.at[idx_v.at[pl.ds(j * G, G)]],
                     rows_v.at[b], gsem.at[b])

  def drain_out(b):
    pltpu.make_async_copy(
        out_hbm.at[pl.ds(base, G)], rows_v.at[b], osem.at[b]).wait()

  pltpu.sync_copy(idx_hbm.at[pl.ds(base, B_PER_W)], idx_v)

  for j in range(NBUF - 1):
    fire_gather(j, j)

  def outer(s, carry):
    for b in range(NBUF):
      i = s * NBUF + b
      pltpu.make_async_copy(
          table_hbm.at[idx_v.at[pl.ds(0, G)]], rows_v.at[b], gsem.at[b]).wait()
      pltpu.async_copy(
          rows_v.at[b], out_hbm.at[pl.ds(base + i * G, G)], osem.at[b])
      bn = (b + NBUF - 1) % NBUF
      if b == 0:
        @pl.when(s > 0)
        def _():
          drain_out(bn)
        fire_gather(i + NBUF - 1, bn)
      else:
        @pl.when(s < S_OUTER - 1)
        def _():
          drain_out(bn)
          fire_gather(i + NBUF - 1, bn)
    return carry

  lax.fori_loop(0, S_OUTER, outer, 0)

  for b in range(NBUF):
    drain_out(b)


def kernel(input_, weight):
  idx = input_.T.reshape(B_TOTAL).astype(jnp.int32)
  out = _gather_kernel(idx, weight)
  return out.reshape(HIST, BATCH, EMB).transpose(1, 0, 2)
